# baseline (speedup 1.0000x reference)
import functools

import jax
import jax.numpy as jnp
from jax import lax
from jax.experimental import pallas as pl
from jax.experimental.pallas import tpu as pltpu
from jax.experimental.pallas import tpu_sc as plsc

N_NODES = 10000
D_FEAT = 256
N_EDGES = 160000

NC = 2
NS = 16
NW = NC * NS
L = 16

C = 128
_CHUNKS_PER_SET = -(-N_EDGES // (NW * C))
EPW = _CHUNKS_PER_SET * C
EPAD = EPW * NW
M = 2 * EPAD
EPW2 = 2 * EPW
K2 = 2 * _CHUNKS_PER_SET


def _sc_body(z_hbm, src_hbm, dst_hbm, out_hbm,
             sidx, didx, s0, s1, d0, d1, lbuf, tbuf, sem0, sem1):
    wid = lax.axis_index("s") * NC + lax.axis_index("c")
    base = wid * EPW2

    pltpu.sync_copy(src_hbm.at[pl.ds(base, EPW2)], sidx)
    pltpu.sync_copy(dst_hbm.at[pl.ds(base, EPW2)], didx)

    bufs = ((s0, d0, sem0), (s1, d1, sem1))

    def issue(k, b):
        sb, db, sem = bufs[b]
        pltpu.make_async_copy(
            z_hbm.at[sidx.at[pl.ds(k * C, C)]], sb, sem).start()
        pltpu.make_async_copy(
            z_hbm.at[didx.at[pl.ds(k * C, C)]], db, sem).start()

    def wait(b):
        sb, db, sem = bufs[b]
        pltpu.make_async_copy(
            z_hbm.at[sidx.at[pl.ds(0, C)]], sb, sem).wait()
        pltpu.make_async_copy(
            z_hbm.at[didx.at[pl.ds(0, C)]], db, sem).wait()

    def compute(k, b):
        sb, db, _ = bufs[b]
        lane16 = lax.iota(jnp.int32, L) * L

        def group(q, carry):
            e0 = q * L
            for r in range(L):
                e = e0 + r
                accs = [None] * 4
                for g in range(8):
                    vs = plsc.bitcast(sb[e, pl.ds(g * L, L)], jnp.bfloat16)
                    vd = plsc.bitcast(db[e, pl.ds(g * L, L)], jnp.bfloat16)
                    s_lo, s_hi = plsc.unpack(
                        vs, format=plsc.PackFormat.INTERLEAVED)
                    d_lo, d_hi = plsc.unpack(
                        vd, format=plsc.PackFormat.INTERLEAVED)
                    jlo = g % 4
                    plo = s_lo * d_lo
                    phi = s_hi * d_hi
                    accs[jlo] = plo if accs[jlo] is None else accs[jlo] + plo
                    accs[jlo] = accs[jlo] + phi
                tbuf[pl.ds(r * L, L)] = (
                    (accs[0] + accs[1]) + (accs[2] + accs[3]))
            vals = jnp.zeros((L,), jnp.float32)
            for col in range(L):
                vals = vals + plsc.load_gather(tbuf, [lane16 + col])
            lbuf[pl.ds(k * C + e0, L)] = vals
            return carry

        lax.fori_loop(0, C // L, group, 0)

    issue(0, 0)
    issue(1, 1)

    def outer(i, carry):
        g = i * 2
        for b in range(2):
            k = g + b
            wait(b)
            compute(k, b)

            @pl.when(k + 2 < K2)
            def _():
                issue(k + 2, b)
        return carry

    lax.fori_loop(0, K2 // 2, outer, 0)

    pltpu.sync_copy(lbuf, out_hbm.at[pl.ds(base, EPW2)])


_sc_gather_dot = functools.partial(
    pl.kernel,
    out_type=jax.ShapeDtypeStruct((M,), jnp.float32),
    mesh=plsc.VectorSubcoreMesh(core_axis_name="c", subcore_axis_name="s"),
    compiler_params=pltpu.CompilerParams(needs_layout_passes=False),
    scratch_types=[
        pltpu.VMEM((EPW2,), jnp.int32),
        pltpu.VMEM((EPW2,), jnp.int32),
        pltpu.VMEM((C, D_FEAT // 2), jnp.int32),
        pltpu.VMEM((C, D_FEAT // 2), jnp.int32),
        pltpu.VMEM((C, D_FEAT // 2), jnp.int32),
        pltpu.VMEM((C, D_FEAT // 2), jnp.int32),
        pltpu.VMEM((EPW2,), jnp.float32),
        pltpu.VMEM((L * L,), jnp.float32),
        pltpu.SemaphoreType.DMA,
        pltpu.SemaphoreType.DMA,
    ],
)(_sc_body)


def _loss_body(pos_ref, neg_ref, out_ref):
    eps = 1e-15
    x = pos_ref[...]
    s = 1.0 / (1.0 + jnp.exp(-x))
    pos_loss = -jnp.sum(jnp.log(s + eps)) / N_EDGES
    y = neg_ref[...]
    t = 1.0 / (1.0 + jnp.exp(-y))
    neg_loss = -jnp.sum(jnp.log(1.0 - t + eps)) / N_EDGES
    out_ref[0, 0] = pos_loss + neg_loss


_loss_reduce = pl.pallas_call(
    _loss_body,
    out_shape=jax.ShapeDtypeStruct((1, 1), jnp.float32),
    out_specs=pl.BlockSpec(memory_space=pltpu.SMEM),
)


def kernel(z, pos_edge_index, neg_edge_index):
    z16 = z.astype(jnp.bfloat16)
    zpacked = lax.bitcast_convert_type(
        z16.reshape(N_NODES, D_FEAT // 2, 2), jnp.int32)
    pad = jnp.zeros((EPAD - N_EDGES,), jnp.int32)
    srcs = jnp.concatenate(
        [pos_edge_index[0], pad, neg_edge_index[0], pad])
    dsts = jnp.concatenate(
        [pos_edge_index[1], pad, neg_edge_index[1], pad])
    logits = _sc_gather_dot(zpacked, srcs, dsts)
    pos_logits = logits[:N_EDGES].reshape(1250, 128)
    neg_logits = logits[EPAD:EPAD + N_EDGES].reshape(1250, 128)
    loss = _loss_reduce(pos_logits, neg_logits)
    return loss[0, 0]


# speedup vs baseline: 2.4552x; 2.4552x over previous
import functools

import jax
import jax.numpy as jnp
from jax import lax
from jax.experimental import pallas as pl
from jax.experimental.pallas import tpu as pltpu
from jax.experimental.pallas import tpu_sc as plsc

N_NODES = 10000
D_FEAT = 256
N_EDGES = 160000

NC = 2
NS = 16
NW = NC * NS
L = 16

C = 32
_CHUNKS_PER_SET = -(-N_EDGES // (NW * C))
EPW = _CHUNKS_PER_SET * C
EPAD = EPW * NW
M = 2 * EPAD
EPW2 = 2 * EPW
K2 = 2 * _CHUNKS_PER_SET


def _sc_body(z_hbm, src_hbm, dst_hbm, out_hbm,
             zsp, sidx, didx, s0, s1, d0, d1, lbuf, tbuf, sem0, sem1):
    wid = lax.axis_index("s") * NC + lax.axis_index("c")
    base = wid * EPW2

    pltpu.sync_copy(src_hbm.at[pl.ds(base, EPW2)], sidx)
    pltpu.sync_copy(dst_hbm.at[pl.ds(base, EPW2)], didx)

    @pl.when(lax.axis_index("s") == 0)
    def _():
        pltpu.sync_copy(z_hbm, zsp)

    plsc.subcore_barrier()

    bufs = ((s0, d0, sem0), (s1, d1, sem1))

    def issue(k, b):
        sb, db, sem = bufs[b]
        pltpu.make_async_copy(
            zsp.at[sidx.at[pl.ds(k * C, C)]], sb, sem).start()
        pltpu.make_async_copy(
            zsp.at[didx.at[pl.ds(k * C, C)]], db, sem).start()

    def wait(b):
        sb, db, sem = bufs[b]
        pltpu.make_async_copy(
            zsp.at[sidx.at[pl.ds(0, C)]], sb, sem).wait()
        pltpu.make_async_copy(
            zsp.at[didx.at[pl.ds(0, C)]], db, sem).wait()

    def compute(k, b):
        sb, db, _ = bufs[b]
        lane16 = lax.iota(jnp.int32, L) * L

        def group(q, carry):
            e0 = q * L
            for r in range(L):
                e = e0 + r
                accs = [None] * 4
                for g in range(8):
                    vs = plsc.bitcast(sb[e, pl.ds(g * L, L)], jnp.bfloat16)
                    vd = plsc.bitcast(db[e, pl.ds(g * L, L)], jnp.bfloat16)
                    s_lo, s_hi = plsc.unpack(
                        vs, format=plsc.PackFormat.INTERLEAVED)
                    d_lo, d_hi = plsc.unpack(
                        vd, format=plsc.PackFormat.INTERLEAVED)
                    jlo = g % 4
                    plo = s_lo * d_lo
                    phi = s_hi * d_hi
                    accs[jlo] = plo if accs[jlo] is None else accs[jlo] + plo
                    accs[jlo] = accs[jlo] + phi
                tbuf[pl.ds(r * L, L)] = (
                    (accs[0] + accs[1]) + (accs[2] + accs[3]))
            vals = jnp.zeros((L,), jnp.float32)
            for col in range(L):
                vals = vals + plsc.load_gather(tbuf, [lane16 + col])
            lbuf[pl.ds(k * C + e0, L)] = vals
            return carry

        lax.fori_loop(0, C // L, group, 0)

    issue(0, 0)
    issue(1, 1)

    def outer(i, carry):
        g = i * 2
        for b in range(2):
            k = g + b
            wait(b)
            compute(k, b)

            @pl.when(k + 2 < K2)
            def _():
                issue(k + 2, b)
        return carry

    lax.fori_loop(0, K2 // 2, outer, 0)

    pltpu.sync_copy(lbuf, out_hbm.at[pl.ds(base, EPW2)])


_sc_gather_dot = functools.partial(
    pl.kernel,
    out_type=jax.ShapeDtypeStruct((M,), jnp.float32),
    mesh=plsc.VectorSubcoreMesh(core_axis_name="c", subcore_axis_name="s"),
    compiler_params=pltpu.CompilerParams(needs_layout_passes=False),
    scratch_types=[
        pltpu.VMEM_SHARED((N_NODES, D_FEAT // 2), jnp.int32),
        pltpu.VMEM((EPW2,), jnp.int32),
        pltpu.VMEM((EPW2,), jnp.int32),
        pltpu.VMEM((C, D_FEAT // 2), jnp.int32),
        pltpu.VMEM((C, D_FEAT // 2), jnp.int32),
        pltpu.VMEM((C, D_FEAT // 2), jnp.int32),
        pltpu.VMEM((C, D_FEAT // 2), jnp.int32),
        pltpu.VMEM((EPW2,), jnp.float32),
        pltpu.VMEM((L * L,), jnp.float32),
        pltpu.SemaphoreType.DMA,
        pltpu.SemaphoreType.DMA,
    ],
)(_sc_body)


def _loss_body(pos_ref, neg_ref, out_ref):
    eps = 1e-15
    x = pos_ref[...]
    s = 1.0 / (1.0 + jnp.exp(-x))
    pos_loss = -jnp.sum(jnp.log(s + eps)) / N_EDGES
    y = neg_ref[...]
    t = 1.0 / (1.0 + jnp.exp(-y))
    neg_loss = -jnp.sum(jnp.log(1.0 - t + eps)) / N_EDGES
    out_ref[0, 0] = pos_loss + neg_loss


_loss_reduce = pl.pallas_call(
    _loss_body,
    out_shape=jax.ShapeDtypeStruct((1, 1), jnp.float32),
    out_specs=pl.BlockSpec(memory_space=pltpu.SMEM),
)


def kernel(z, pos_edge_index, neg_edge_index):
    z16 = z.astype(jnp.bfloat16)
    zpacked = lax.bitcast_convert_type(
        z16.reshape(N_NODES, D_FEAT // 2, 2), jnp.int32)
    pad = jnp.zeros((EPAD - N_EDGES,), jnp.int32)
    srcs = jnp.concatenate(
        [pos_edge_index[0], pad, neg_edge_index[0], pad])
    dsts = jnp.concatenate(
        [pos_edge_index[1], pad, neg_edge_index[1], pad])
    logits = _sc_gather_dot(zpacked, srcs, dsts)
    pos_logits = logits[:N_EDGES].reshape(1250, 128)
    neg_logits = logits[EPAD:EPAD + N_EDGES].reshape(1250, 128)
    loss = _loss_reduce(pos_logits, neg_logits)
    return loss[0, 0]


# speedup vs baseline: 2.4628x; 1.0031x over previous
import functools

import jax
import jax.numpy as jnp
from jax import lax
from jax.experimental import pallas as pl
from jax.experimental.pallas import tpu as pltpu
from jax.experimental.pallas import tpu_sc as plsc

N_NODES = 10000
D_FEAT = 256
N_EDGES = 160000

NC = 2
NS = 16
NW = NC * NS
L = 16

C = 32
_CHUNKS_PER_SET = -(-N_EDGES // (NW * C))
EPW = _CHUNKS_PER_SET * C
EPAD = EPW * NW
M = 2 * EPAD
EPW2 = 2 * EPW
K2 = 2 * _CHUNKS_PER_SET


def _sc_body(z_hbm, src_hbm, dst_hbm, out_hbm,
             zsp, sidx, didx, s0, s1, d0, d1, lbuf, tbuf, sem0, sem1):
    wid = lax.axis_index("s") * NC + lax.axis_index("c")
    base = wid * EPW2

    pltpu.sync_copy(src_hbm.at[pl.ds(base, EPW2)], sidx)
    pltpu.sync_copy(dst_hbm.at[pl.ds(base, EPW2)], didx)

    @pl.when(lax.axis_index("s") == 0)
    def _():
        pltpu.sync_copy(z_hbm, zsp)

    plsc.subcore_barrier()

    bufs = ((s0, d0, sem0), (s1, d1, sem1))

    def issue(k, b):
        sb, db, sem = bufs[b]
        pltpu.make_async_copy(
            zsp.at[sidx.at[pl.ds(k * C, C)]], sb, sem).start()
        pltpu.make_async_copy(
            zsp.at[didx.at[pl.ds(k * C, C)]], db, sem).start()

    def wait(b):
        sb, db, sem = bufs[b]
        pltpu.make_async_copy(
            zsp.at[sidx.at[pl.ds(0, C)]], sb, sem).wait()
        pltpu.make_async_copy(
            zsp.at[didx.at[pl.ds(0, C)]], db, sem).wait()

    def compute(k, b):
        sb, db, _ = bufs[b]
        lane16 = lax.iota(jnp.int32, L) * L

        def group(q, carry):
            e0 = q * L
            for r in range(L):
                e = e0 + r
                accs = [None] * 4
                for g in range(8):
                    vs = plsc.bitcast(sb[e, pl.ds(g * L, L)], jnp.bfloat16)
                    vd = plsc.bitcast(db[e, pl.ds(g * L, L)], jnp.bfloat16)
                    p_lo, p_hi = plsc.unpack(
                        vs * vd, format=plsc.PackFormat.INTERLEAVED)
                    j0 = 2 * (g % 2)
                    accs[j0] = p_lo if accs[j0] is None else accs[j0] + p_lo
                    accs[j0 + 1] = (
                        p_hi if accs[j0 + 1] is None else accs[j0 + 1] + p_hi)
                tbuf[pl.ds(r * L, L)] = (
                    (accs[0] + accs[1]) + (accs[2] + accs[3]))
            vals = jnp.zeros((L,), jnp.float32)
            for col in range(L):
                vals = vals + plsc.load_gather(tbuf, [lane16 + col])
            lbuf[pl.ds(k * C + e0, L)] = vals
            return carry

        lax.fori_loop(0, C // L, group, 0)

    issue(0, 0)
    issue(1, 1)

    def outer(i, carry):
        g = i * 2
        for b in range(2):
            k = g + b
            wait(b)
            compute(k, b)

            @pl.when(k + 2 < K2)
            def _():
                issue(k + 2, b)
        return carry

    lax.fori_loop(0, K2 // 2, outer, 0)

    pltpu.sync_copy(lbuf, out_hbm.at[pl.ds(base, EPW2)])


_sc_gather_dot = functools.partial(
    pl.kernel,
    out_type=jax.ShapeDtypeStruct((M,), jnp.float32),
    mesh=plsc.VectorSubcoreMesh(core_axis_name="c", subcore_axis_name="s"),
    compiler_params=pltpu.CompilerParams(needs_layout_passes=False),
    scratch_types=[
        pltpu.VMEM_SHARED((N_NODES, D_FEAT // 2), jnp.int32),
        pltpu.VMEM((EPW2,), jnp.int32),
        pltpu.VMEM((EPW2,), jnp.int32),
        pltpu.VMEM((C, D_FEAT // 2), jnp.int32),
        pltpu.VMEM((C, D_FEAT // 2), jnp.int32),
        pltpu.VMEM((C, D_FEAT // 2), jnp.int32),
        pltpu.VMEM((C, D_FEAT // 2), jnp.int32),
        pltpu.VMEM((EPW2,), jnp.float32),
        pltpu.VMEM((L * L,), jnp.float32),
        pltpu.SemaphoreType.DMA,
        pltpu.SemaphoreType.DMA,
    ],
)(_sc_body)


def _loss_body(pos_ref, neg_ref, out_ref):
    eps = 1e-15
    x = pos_ref[...]
    s = 1.0 / (1.0 + jnp.exp(-x))
    pos_loss = -jnp.sum(jnp.log(s + eps)) / N_EDGES
    y = neg_ref[...]
    t = 1.0 / (1.0 + jnp.exp(-y))
    neg_loss = -jnp.sum(jnp.log(1.0 - t + eps)) / N_EDGES
    out_ref[0, 0] = pos_loss + neg_loss


_loss_reduce = pl.pallas_call(
    _loss_body,
    out_shape=jax.ShapeDtypeStruct((1, 1), jnp.float32),
    out_specs=pl.BlockSpec(memory_space=pltpu.SMEM),
)


def kernel(z, pos_edge_index, neg_edge_index):
    z16 = z.astype(jnp.bfloat16)
    zpacked = lax.bitcast_convert_type(
        z16.reshape(N_NODES, D_FEAT // 2, 2), jnp.int32)
    pad = jnp.zeros((EPAD - N_EDGES,), jnp.int32)
    srcs = jnp.concatenate(
        [pos_edge_index[0], pad, neg_edge_index[0], pad])
    dsts = jnp.concatenate(
        [pos_edge_index[1], pad, neg_edge_index[1], pad])
    logits = _sc_gather_dot(zpacked, srcs, dsts)
    pos_logits = logits[:N_EDGES].reshape(1250, 128)
    neg_logits = logits[EPAD:EPAD + N_EDGES].reshape(1250, 128)
    loss = _loss_reduce(pos_logits, neg_logits)
    return loss[0, 0]


# speedup vs baseline: 3.5184x; 1.4286x over previous
import functools

import jax
import jax.numpy as jnp
from jax import lax
from jax.experimental import pallas as pl
from jax.experimental.pallas import tpu as pltpu
from jax.experimental.pallas import tpu_sc as plsc

N_NODES = 10000
D_FEAT = 256
N_EDGES = 160000

NC = 2
NS = 16
NW = NC * NS
L = 16

C = 32
_CHUNKS_PER_SET = -(-N_EDGES // (NW * C))
EPW = _CHUNKS_PER_SET * C
EPAD = EPW * NW
M = 2 * EPAD
EPW2 = 2 * EPW
K2 = 2 * _CHUNKS_PER_SET


def _sc_body(z_hbm, src_hbm, dst_hbm, out_hbm,
             zsp, sidx, didx, s0, s1, d0, d1, lbuf, sem0, sem1):
    wid = lax.axis_index("s") * NC + lax.axis_index("c")
    base = wid * EPW2

    pltpu.sync_copy(src_hbm.at[pl.ds(base, EPW2)], sidx)
    pltpu.sync_copy(dst_hbm.at[pl.ds(base, EPW2)], didx)

    @pl.when(lax.axis_index("s") == 0)
    def _():
        pltpu.sync_copy(z_hbm, zsp)

    plsc.subcore_barrier()

    bufs = ((s0, d0, sem0), (s1, d1, sem1))

    def issue(k, b):
        sb, db, sem = bufs[b]
        pltpu.make_async_copy(
            zsp.at[sidx.at[pl.ds(k * C, C)]], sb, sem).start()
        pltpu.make_async_copy(
            zsp.at[didx.at[pl.ds(k * C, C)]], db, sem).start()

    def wait(b):
        sb, db, sem = bufs[b]
        pltpu.make_async_copy(
            zsp.at[sidx.at[pl.ds(0, C)]], sb, sem).wait()
        pltpu.make_async_copy(
            zsp.at[didx.at[pl.ds(0, C)]], db, sem).wait()

    def compute(k, b):
        sb, db, _ = bufs[b]
        lane = lax.iota(jnp.int32, L)

        @plsc.parallel_loop(0, C // L, unroll=2)
        def group(q):
            e0 = q * L
            vals = jnp.zeros((L,), jnp.float32)
            for r in range(L):
                e = e0 + r
                accs = [None] * 4
                for g in range(8):
                    vs = plsc.bitcast(sb[e, pl.ds(g * L, L)], jnp.bfloat16)
                    vd = plsc.bitcast(db[e, pl.ds(g * L, L)], jnp.bfloat16)
                    p_lo, p_hi = plsc.unpack(
                        vs * vd, format=plsc.PackFormat.INTERLEAVED)
                    j0 = 2 * (g % 2)
                    accs[j0] = p_lo if accs[j0] is None else accs[j0] + p_lo
                    accs[j0 + 1] = (
                        p_hi if accs[j0 + 1] is None else accs[j0 + 1] + p_hi)
                tot = (accs[0] + accs[1]) + (accs[2] + accs[3])
                vals = jnp.where(lane == r, jnp.sum(tot), vals)
            lbuf[pl.ds(k * C + e0, L)] = vals

    issue(0, 0)
    issue(1, 1)

    def outer(i, carry):
        g = i * 2
        for b in range(2):
            k = g + b
            wait(b)
            compute(k, b)

            @pl.when(k + 2 < K2)
            def _():
                issue(k + 2, b)
        return carry

    lax.fori_loop(0, K2 // 2, outer, 0)

    pltpu.sync_copy(lbuf, out_hbm.at[pl.ds(base, EPW2)])


_sc_gather_dot = functools.partial(
    pl.kernel,
    out_type=jax.ShapeDtypeStruct((M,), jnp.float32),
    mesh=plsc.VectorSubcoreMesh(core_axis_name="c", subcore_axis_name="s"),
    compiler_params=pltpu.CompilerParams(needs_layout_passes=False),
    scratch_types=[
        pltpu.VMEM_SHARED((N_NODES, D_FEAT // 2), jnp.int32),
        pltpu.VMEM((EPW2,), jnp.int32),
        pltpu.VMEM((EPW2,), jnp.int32),
        pltpu.VMEM((C, D_FEAT // 2), jnp.int32),
        pltpu.VMEM((C, D_FEAT // 2), jnp.int32),
        pltpu.VMEM((C, D_FEAT // 2), jnp.int32),
        pltpu.VMEM((C, D_FEAT // 2), jnp.int32),
        pltpu.VMEM((EPW2,), jnp.float32),
        pltpu.SemaphoreType.DMA,
        pltpu.SemaphoreType.DMA,
    ],
)(_sc_body)


def _loss_body(pos_ref, neg_ref, out_ref):
    eps = 1e-15
    x = pos_ref[...]
    s = 1.0 / (1.0 + jnp.exp(-x))
    pos_loss = -jnp.sum(jnp.log(s + eps)) / N_EDGES
    y = neg_ref[...]
    t = 1.0 / (1.0 + jnp.exp(-y))
    neg_loss = -jnp.sum(jnp.log(1.0 - t + eps)) / N_EDGES
    out_ref[0, 0] = pos_loss + neg_loss


_loss_reduce = pl.pallas_call(
    _loss_body,
    out_shape=jax.ShapeDtypeStruct((1, 1), jnp.float32),
    out_specs=pl.BlockSpec(memory_space=pltpu.SMEM),
)


def kernel(z, pos_edge_index, neg_edge_index):
    z16 = z.astype(jnp.bfloat16)
    zpacked = lax.bitcast_convert_type(
        z16.reshape(N_NODES, D_FEAT // 2, 2), jnp.int32)
    pad = jnp.zeros((EPAD - N_EDGES,), jnp.int32)
    srcs = jnp.concatenate(
        [pos_edge_index[0], pad, neg_edge_index[0], pad])
    dsts = jnp.concatenate(
        [pos_edge_index[1], pad, neg_edge_index[1], pad])
    logits = _sc_gather_dot(zpacked, srcs, dsts)
    pos_logits = logits[:N_EDGES].reshape(1250, 128)
    neg_logits = logits[EPAD:EPAD + N_EDGES].reshape(1250, 128)
    loss = _loss_reduce(pos_logits, neg_logits)
    return loss[0, 0]


# speedup vs baseline: 3.5512x; 1.0093x over previous
import functools

import jax
import jax.numpy as jnp
from jax import lax
from jax.experimental import pallas as pl
from jax.experimental.pallas import tpu as pltpu
from jax.experimental.pallas import tpu_sc as plsc

N_NODES = 10000
D_FEAT = 256
N_EDGES = 160000

NC = 2
NS = 16
NW = NC * NS
L = 16

C = 64
_CHUNKS_PER_SET = -(-N_EDGES // (NW * C))
EPW = _CHUNKS_PER_SET * C
EPAD = EPW * NW
M = 2 * EPAD
EPW2 = 2 * EPW
K2 = 2 * _CHUNKS_PER_SET


def _sc_body(z_hbm, eidx_hbm, out_hbm,
             zsp, eidx, si0, si1, di0, di1, o0, o1,
             r0, r1, r2, r3, semg0, semg1, semo0, semo1):
    wid = lax.axis_index("s") * NC + lax.axis_index("c")
    base = wid * EPW2

    pltpu.sync_copy(eidx_hbm.at[pl.ds(base, EPW2)], eidx)

    @pl.when(lax.axis_index("s") == 0)
    def _():
        pltpu.sync_copy(z_hbm, zsp)

    plsc.subcore_barrier()

    bufs = (
        (si0, di0, o0, (r0, r1), semg0, semo0),
        (si1, di1, o1, (r2, r3), semg1, semo1),
    )

    def unpack_idx(k, b):
        sib, dib = bufs[b][0], bufs[b][1]
        for w in range(C // L):
            v = eidx[pl.ds(k * C + w * L, L)]
            sib[pl.ds(w * L, L)] = v & 0xFFFF
            dib[pl.ds(w * L, L)] = lax.shift_right_logical(v, 16)

    def issue(k, b):
        sib, dib, _, (rs, rd), semg, _ = bufs[b]
        unpack_idx(k, b)
        pltpu.make_async_copy(zsp.at[sib], rs, semg).start()
        pltpu.make_async_copy(zsp.at[dib], rd, semg).start()

    def wait_gather(b):
        sib, dib, _, (rs, rd), semg, _ = bufs[b]
        pltpu.make_async_copy(zsp.at[sib], rs, semg).wait()
        pltpu.make_async_copy(zsp.at[dib], rd, semg).wait()

    def compute(k, b):
        _, _, ob, (rs, rd), _, _ = bufs[b]
        lane = lax.iota(jnp.int32, L)

        @plsc.parallel_loop(0, C // L, unroll=2)
        def group(q):
            e0 = q * L
            vals = jnp.zeros((L,), jnp.float32)
            for r in range(L):
                e = e0 + r
                accs = [None] * 4
                for g in range(8):
                    vs = plsc.bitcast(rs[e, pl.ds(g * L, L)], jnp.bfloat16)
                    vd = plsc.bitcast(rd[e, pl.ds(g * L, L)], jnp.bfloat16)
                    p_lo, p_hi = plsc.unpack(
                        vs * vd, format=plsc.PackFormat.INTERLEAVED)
                    j0 = 2 * (g % 2)
                    accs[j0] = p_lo if accs[j0] is None else accs[j0] + p_lo
                    accs[j0 + 1] = (
                        p_hi if accs[j0 + 1] is None else accs[j0 + 1] + p_hi)
                tot = (accs[0] + accs[1]) + (accs[2] + accs[3])
                vals = jnp.where(lane == r, jnp.sum(tot), vals)
            ob[pl.ds(e0, L)] = vals

    def start_out(k, b):
        ob, semo = bufs[b][2], bufs[b][5]
        pltpu.make_async_copy(
            ob, out_hbm.at[pl.ds(base + k * C, C)], semo).start()

    def wait_out(b):
        ob, semo = bufs[b][2], bufs[b][5]
        pltpu.make_async_copy(
            ob, out_hbm.at[pl.ds(base, C)], semo).wait()

    issue(0, 0)
    issue(1, 1)

    def outer(i, carry):
        g = i * 2
        for b in range(2):
            k = g + b
            wait_gather(b)

            @pl.when(k >= 2)
            def _():
                wait_out(b)

            compute(k, b)
            start_out(k, b)

            @pl.when(k + 2 < K2)
            def _():
                issue(k + 2, b)
        return carry

    lax.fori_loop(0, K2 // 2, outer, 0)

    wait_out(0)
    wait_out(1)


_sc_gather_dot = functools.partial(
    pl.kernel,
    out_type=jax.ShapeDtypeStruct((M,), jnp.float32),
    mesh=plsc.VectorSubcoreMesh(core_axis_name="c", subcore_axis_name="s"),
    compiler_params=pltpu.CompilerParams(needs_layout_passes=False),
    scratch_types=[
        pltpu.VMEM_SHARED((N_NODES, D_FEAT // 2), jnp.int32),
        pltpu.VMEM((EPW2,), jnp.int32),
        pltpu.VMEM((C,), jnp.int32),
        pltpu.VMEM((C,), jnp.int32),
        pltpu.VMEM((C,), jnp.int32),
        pltpu.VMEM((C,), jnp.int32),
        pltpu.VMEM((C,), jnp.float32),
        pltpu.VMEM((C,), jnp.float32),
        pltpu.VMEM((C, D_FEAT // 2), jnp.int32),
        pltpu.VMEM((C, D_FEAT // 2), jnp.int32),
        pltpu.VMEM((C, D_FEAT // 2), jnp.int32),
        pltpu.VMEM((C, D_FEAT // 2), jnp.int32),
        pltpu.SemaphoreType.DMA,
        pltpu.SemaphoreType.DMA,
        pltpu.SemaphoreType.DMA,
        pltpu.SemaphoreType.DMA,
    ],
)(_sc_body)


def _loss_body(pos_ref, neg_ref, out_ref):
    eps = 1e-15
    x = pos_ref[...]
    s = 1.0 / (1.0 + jnp.exp(-x))
    pos_loss = -jnp.sum(jnp.log(s + eps)) / N_EDGES
    y = neg_ref[...]
    t = 1.0 / (1.0 + jnp.exp(-y))
    neg_loss = -jnp.sum(jnp.log(1.0 - t + eps)) / N_EDGES
    out_ref[0, 0] = pos_loss + neg_loss


_loss_reduce = pl.pallas_call(
    _loss_body,
    out_shape=jax.ShapeDtypeStruct((1, 1), jnp.float32),
    out_specs=pl.BlockSpec(memory_space=pltpu.SMEM),
)


def kernel(z, pos_edge_index, neg_edge_index):
    z16 = z.astype(jnp.bfloat16)
    zpacked = lax.bitcast_convert_type(
        z16.reshape(N_NODES, D_FEAT // 2, 2), jnp.int32)
    pad = jnp.zeros((EPAD - N_EDGES,), jnp.int32)
    pos_packed = pos_edge_index[0] | (pos_edge_index[1] << 16)
    neg_packed = neg_edge_index[0] | (neg_edge_index[1] << 16)
    eidx = jnp.concatenate([pos_packed, pad, neg_packed, pad])
    logits = _sc_gather_dot(zpacked, eidx)
    pos_logits = logits[:N_EDGES].reshape(1250, 128)
    neg_logits = logits[EPAD:EPAD + N_EDGES].reshape(1250, 128)
    loss = _loss_reduce(pos_logits, neg_logits)
    return loss[0, 0]
